# baseline (speedup 1.0000x reference)
import functools

import jax
import jax.numpy as jnp
from jax.experimental import pallas as pl


def _mm_kernel(x_ref, w_ref, o_ref):
    o_ref[...] = jnp.dot(x_ref[...], w_ref[...],
                         preferred_element_type=jnp.float32)


def _layer1_kernel(adj_ref, s1_ref, w2_ref, o_ref):
    t = jnp.dot(adj_ref[...], s1_ref[...], preferred_element_type=jnp.float32)
    h = jnp.maximum(t, 0.0)
    o_ref[...] = jnp.dot(h, w2_ref[...], preferred_element_type=jnp.float32)


def _layer2_kernel(adj_ref, s2_ref, o_ref):
    o_ref[...] = jnp.dot(adj_ref[...], s2_ref[...],
                         preferred_element_type=jnp.float32)


@functools.partial(jax.jit, static_argnames=())
def kernel(x, adj, W1, W2):
    n, nfeat = x.shape
    nhid = W1.shape[1]
    nclass = W2.shape[1]

    bm = 400
    n_blocks = n // bm

    s1 = pl.pallas_call(
        _mm_kernel,
        grid=(10,),
        in_specs=[
            pl.BlockSpec((n // 10, nfeat), lambda i: (i, 0)),
            pl.BlockSpec((nfeat, nhid), lambda i: (0, 0)),
        ],
        out_specs=pl.BlockSpec((n // 10, nhid), lambda i: (i, 0)),
        out_shape=jax.ShapeDtypeStruct((n, nhid), jnp.float32),
    )(x, W1)

    s2 = pl.pallas_call(
        _layer1_kernel,
        grid=(n_blocks,),
        in_specs=[
            pl.BlockSpec((bm, n), lambda i: (i, 0)),
            pl.BlockSpec((n, nhid), lambda i: (0, 0)),
            pl.BlockSpec((nhid, nclass), lambda i: (0, 0)),
        ],
        out_specs=pl.BlockSpec((bm, nclass), lambda i: (i, 0)),
        out_shape=jax.ShapeDtypeStruct((n, nclass), jnp.float32),
    )(adj, s1, W2)

    out = pl.pallas_call(
        _layer2_kernel,
        grid=(n_blocks,),
        in_specs=[
            pl.BlockSpec((bm, n), lambda i: (i, 0)),
            pl.BlockSpec((n, nclass), lambda i: (0, 0)),
        ],
        out_specs=pl.BlockSpec((bm, nclass), lambda i: (i, 0)),
        out_shape=jax.ShapeDtypeStruct((n, nclass), jnp.float32),
    )(adj, s2)

    return out


# speedup vs baseline: 1.0020x; 1.0020x over previous
import functools

import jax
import jax.numpy as jnp
from jax.experimental import pallas as pl


def _mm_kernel(x_ref, w_ref, o_ref):
    o_ref[...] = jnp.dot(x_ref[...], w_ref[...],
                         preferred_element_type=jnp.float32)


def _layer1_kernel(adj_ref, s1_ref, w2_ref, o_ref):
    t = jnp.dot(adj_ref[...], s1_ref[...], preferred_element_type=jnp.float32,
                precision=jax.lax.Precision.DEFAULT)
    h = jnp.maximum(t, 0.0)
    o_ref[...] = jnp.dot(h, w2_ref[...], preferred_element_type=jnp.float32)


def _layer2_kernel(adj_ref, s2_ref, o_ref):
    o_ref[...] = jnp.dot(adj_ref[...], s2_ref[...],
                         preferred_element_type=jnp.float32,
                         precision=jax.lax.Precision.DEFAULT)


@functools.partial(jax.jit, static_argnames=())
def kernel(x, adj, W1, W2):
    n, nfeat = x.shape
    nhid = W1.shape[1]
    nclass = W2.shape[1]

    bm = 400
    n_blocks = n // bm

    s1 = pl.pallas_call(
        _mm_kernel,
        grid=(10,),
        in_specs=[
            pl.BlockSpec((n // 10, nfeat), lambda i: (i, 0)),
            pl.BlockSpec((nfeat, nhid), lambda i: (0, 0)),
        ],
        out_specs=pl.BlockSpec((n // 10, nhid), lambda i: (i, 0)),
        out_shape=jax.ShapeDtypeStruct((n, nhid), jnp.float32),
    )(x, W1)

    s2 = pl.pallas_call(
        _layer1_kernel,
        grid=(n_blocks,),
        in_specs=[
            pl.BlockSpec((bm, n), lambda i: (i, 0)),
            pl.BlockSpec((n, nhid), lambda i: (0, 0)),
            pl.BlockSpec((nhid, nclass), lambda i: (0, 0)),
        ],
        out_specs=pl.BlockSpec((bm, nclass), lambda i: (i, 0)),
        out_shape=jax.ShapeDtypeStruct((n, nclass), jnp.float32),
    )(adj, s1, W2)

    out = pl.pallas_call(
        _layer2_kernel,
        grid=(n_blocks,),
        in_specs=[
            pl.BlockSpec((bm, n), lambda i: (i, 0)),
            pl.BlockSpec((n, nclass), lambda i: (0, 0)),
        ],
        out_specs=pl.BlockSpec((bm, nclass), lambda i: (i, 0)),
        out_shape=jax.ShapeDtypeStruct((n, nclass), jnp.float32),
    )(adj, s2)

    return out


# speedup vs baseline: 1.0518x; 1.0497x over previous
import functools

import jax
import jax.numpy as jnp
from jax.experimental import pallas as pl
from jax.experimental.pallas import tpu as pltpu

_BM = 400


def _gcn_kernel(x_hbm, w1_ref, w2_ref, adj_ref, o_ref,
                x_vmem, s1_ref, s2_ref, sem, *, nb, bm):
    i = pl.program_id(0)

    @pl.when(i == 0)
    def _():
        copy = pltpu.make_async_copy(x_hbm, x_vmem, sem)
        copy.start()
        copy.wait()
        s1_ref[...] = jnp.dot(x_vmem[...], w1_ref[...],
                              preferred_element_type=jnp.float32)

    @pl.when(i < nb)
    def _():
        t = jnp.dot(adj_ref[...], s1_ref[...],
                    preferred_element_type=jnp.float32)
        h = jnp.maximum(t, 0.0)
        s2_ref[pl.ds(i * bm, bm), :] = jnp.dot(
            h, w2_ref[...], preferred_element_type=jnp.float32)

    @pl.when(i >= nb)
    def _():
        o_ref[...] = jnp.dot(adj_ref[...], s2_ref[...],
                             preferred_element_type=jnp.float32)


def kernel(x, adj, W1, W2):
    n, nfeat = x.shape
    nhid = W1.shape[1]
    nclass = W2.shape[1]
    bm = _BM
    nb = n // bm

    return pl.pallas_call(
        functools.partial(_gcn_kernel, nb=nb, bm=bm),
        grid=(2 * nb,),
        in_specs=[
            pl.BlockSpec(memory_space=pl.ANY),
            pl.BlockSpec((nfeat, nhid), lambda i: (0, 0)),
            pl.BlockSpec((nhid, nclass), lambda i: (0, 0)),
            pl.BlockSpec((bm, n), lambda i: (jax.lax.rem(i, nb), 0)),
        ],
        out_specs=pl.BlockSpec((bm, nclass),
                               lambda i: (jnp.maximum(i - nb, 0), 0)),
        out_shape=jax.ShapeDtypeStruct((n, nclass), jnp.float32),
        scratch_shapes=[
            pltpu.VMEM((n, nfeat), jnp.float32),
            pltpu.VMEM((n, nhid), jnp.float32),
            pltpu.VMEM((n, nclass), jnp.float32),
            pltpu.SemaphoreType.DMA,
        ],
    )(x, W1, W2, adj)


# speedup vs baseline: 1.0690x; 1.0163x over previous
import functools

import jax
import jax.numpy as jnp
from jax.experimental import pallas as pl
from jax.experimental.pallas import tpu as pltpu

_BM = 400


def _gcn_kernel(x_ref, w1_ref, w2_ref, adj_ref, o_ref,
                s1_ref, s2_ref, *, nb, bm):
    i = pl.program_id(0)

    @pl.when(i == 0)
    def _():
        s1_ref[...] = jnp.dot(x_ref[...], w1_ref[...],
                              preferred_element_type=jnp.float32)

    @pl.when(i < nb)
    def _():
        t = jnp.dot(adj_ref[...], s1_ref[...],
                    preferred_element_type=jnp.float32)
        h = jnp.maximum(t, 0.0)
        s2_ref[pl.ds(i * bm, bm), :] = jnp.dot(
            h, w2_ref[...], preferred_element_type=jnp.float32)

    @pl.when(i >= nb)
    def _():
        o_ref[...] = jnp.dot(adj_ref[...], s2_ref[...],
                             preferred_element_type=jnp.float32)


def kernel(x, adj, W1, W2):
    n, nfeat = x.shape
    nhid = W1.shape[1]
    nclass = W2.shape[1]
    bm = _BM
    nb = n // bm

    return pl.pallas_call(
        functools.partial(_gcn_kernel, nb=nb, bm=bm),
        grid=(2 * nb,),
        in_specs=[
            pl.BlockSpec((n, nfeat), lambda i: (0, 0)),
            pl.BlockSpec((nfeat, nhid), lambda i: (0, 0)),
            pl.BlockSpec((nhid, nclass), lambda i: (0, 0)),
            pl.BlockSpec((bm, n), lambda i: (jax.lax.rem(i, nb), 0)),
        ],
        out_specs=pl.BlockSpec((bm, nclass),
                               lambda i: (jnp.maximum(i - nb, 0), 0)),
        out_shape=jax.ShapeDtypeStruct((n, nclass), jnp.float32),
        scratch_shapes=[
            pltpu.VMEM((n, nhid), jnp.float32),
            pltpu.VMEM((n, nclass), jnp.float32),
        ],
    )(x, W1, W2, adj)
